# baseline (speedup 1.0000x reference)
import functools

import numpy as np
import jax
import jax.numpy as jnp
from jax.experimental import pallas as pl
from jax.experimental.pallas import tpu as pltpu

N_EMBD = 1024
N_HEAD = 16
HEAD_DIM = 64
KV_LORA = 256
ROPE_DIM = 32
NOPE_DIM = HEAD_DIM - ROPE_DIM
N_EXP = 8
TOP_K = 2
INTER = 2048
THETA = 100000.0

TBLK = 512
RBLK = 256
EPS = 1e-6


def _rms(x, w):
    return x * jax.lax.rsqrt(jnp.mean(x * x, axis=1, keepdims=True) + EPS) * w


def _silu(x):
    return x / (1.0 + jnp.exp(-x))


def _pre_attn_kernel(x_ref, w1_ref, wq_ref, wkva_ref, wkvb_ref, cos_ref,
                     sin_ref, q_ref, k_ref):
    g = _rms(x_ref[...], w1_ref[...])
    q = jnp.dot(g, wq_ref[...], preferred_element_type=jnp.float32)
    ckv = jnp.dot(g, wkva_ref[...], preferred_element_type=jnp.float32)
    latent = ckv[:, :KV_LORA]
    kr = ckv[:, KV_LORA:]
    kn = jnp.dot(latent, wkvb_ref[...], preferred_element_type=jnp.float32)
    C = cos_ref[...]
    S = sin_ref[...]
    di = jax.lax.broadcasted_iota(jnp.int32, (ROPE_DIM, ROPE_DIM), 0)
    dj = jax.lax.broadcasted_iota(jnp.int32, (ROPE_DIM, ROPE_DIM), 1)
    P = ((di ^ 1) == dj).astype(jnp.float32)

    def rope(v):
        return v * C + jnp.dot(v, P, preferred_element_type=jnp.float32) * S

    kr2 = rope(kr)
    qp = []
    kp = []
    for h in range(N_HEAD):
        qh = q[:, h * HEAD_DIM:(h + 1) * HEAD_DIM]
        qp.append(jnp.concatenate(
            [qh[:, :NOPE_DIM], rope(qh[:, NOPE_DIM:])],
            axis=1).reshape(1, TBLK, HEAD_DIM))
        kp.append(jnp.concatenate(
            [kn[:, h * NOPE_DIM:(h + 1) * NOPE_DIM], kr2],
            axis=1).reshape(1, TBLK, HEAD_DIM))
    q_ref[...] = jnp.concatenate(qp, axis=0)
    k_ref[...] = jnp.concatenate(kp, axis=0)


def _attn_kernel(q_ref, k_ref, o_ref, *, T):
    t = pl.program_id(1)
    q = q_ref[0]
    k = k_ref[0]
    s = jax.lax.dot_general(q, k, (((1,), (1,)), ((), ())),
                            preferred_element_type=jnp.float32)
    s = s * (1.0 / np.float32(np.sqrt(HEAD_DIM)))
    row = jax.lax.broadcasted_iota(jnp.int32, (TBLK, T), 0) + t * TBLK
    col = jax.lax.broadcasted_iota(jnp.int32, (TBLK, T), 1)
    s = jnp.where(col <= row, s, jnp.float32(-1e9))
    m = jnp.max(s, axis=1, keepdims=True)
    e = jnp.exp(s - m)
    p = e * (1.0 / jnp.sum(e, axis=1, keepdims=True))
    o_ref[0] = jnp.dot(p, k, preferred_element_type=jnp.float32)


def _post_attn_kernel(y_ref, wo_ref, x_ref, w2_ref, gate_ref, bias_ref,
                      h_ref, g_ref, logit_ref):
    h = x_ref[...] + jnp.dot(y_ref[...], wo_ref[...],
                             preferred_element_type=jnp.float32)
    h_ref[...] = h
    g = _rms(h, w2_ref[...])
    g_ref[...] = g.astype(jnp.bfloat16)
    logit_ref[...] = (jnp.dot(g, gate_ref[...],
                              preferred_element_type=jnp.float32)
                      + bias_ref[...])


def _route_kernel(logit_ref, s_ref, tok_ref, w_ref, *, T, NPB):
    lg = logit_ref[...]
    m = jnp.max(lg, axis=1, keepdims=True)
    ex = jnp.exp(lg - m)
    probs = ex / jnp.sum(ex, axis=1, keepdims=True)
    io8 = jax.lax.broadcasted_iota(jnp.int32, (T, N_EXP), 1).astype(jnp.float32)
    m1 = jnp.max(probs, axis=1, keepdims=True)
    a1 = jnp.min(jnp.where(probs == m1, io8, jnp.float32(N_EXP)),
                 axis=1, keepdims=True)
    oh1 = (io8 == a1).astype(jnp.float32)
    p2 = jnp.where(oh1 > 0, jnp.float32(-1.0), probs)
    m2 = jnp.max(p2, axis=1, keepdims=True)
    a2 = jnp.min(jnp.where(p2 == m2, io8, jnp.float32(N_EXP)),
                 axis=1, keepdims=True)
    oh2 = (io8 == a2).astype(jnp.float32)
    den = m1 + m2
    w1 = m1 / den
    w2 = m2 / den

    CH = 512
    ci = jax.lax.broadcasted_iota(jnp.int32, (CH, CH), 0)
    cj = jax.lax.broadcasted_iota(jnp.int32, (CH, CH), 1)
    tril = (cj < ci).astype(jnp.float32)

    def exclcum(oh):
        parts = []
        carry = jnp.zeros((1, N_EXP), jnp.float32)
        for c in range(T // CH):
            blk = oh[c * CH:(c + 1) * CH, :]
            parts.append(jnp.dot(tril, blk,
                                 preferred_element_type=jnp.float32) + carry)
            carry = carry + jnp.sum(blk, axis=0, keepdims=True)
        return jnp.concatenate(parts, axis=0), carry

    pos1, tot1 = exclcum(oh1)
    pos2, tot2 = exclcum(oh2)
    counts = tot1 + tot2
    pcnt = jnp.floor((counts + (RBLK - 1)) * (1.0 / RBLK)) * RBLK
    e8i = jax.lax.broadcasted_iota(jnp.int32, (N_EXP, N_EXP), 0)
    e8j = jax.lax.broadcasted_iota(jnp.int32, (N_EXP, N_EXP), 1)
    m8 = (e8i < e8j).astype(jnp.float32)
    pad_off = jnp.dot(pcnt, m8, preferred_element_type=jnp.float32)

    rank1 = jnp.sum(pos1 * oh1, axis=1, keepdims=True)
    rank2 = jnp.sum((pos2 + tot1) * oh2, axis=1, keepdims=True)
    dest1 = jnp.sum(pad_off * oh1, axis=1, keepdims=True) + rank1
    dest2 = jnp.sum(pad_off * oh2, axis=1, keepdims=True) + rank2

    tokc = jax.lax.broadcasted_iota(jnp.int32, (T, 1), 0).astype(jnp.float32)
    riota = jax.lax.broadcasted_iota(jnp.int32, (1, RBLK), 1)

    def body(i, _):
        r = (riota + i * RBLK).astype(jnp.float32)
        mask1 = (dest1 == r).astype(jnp.float32)
        mask2 = (dest2 == r).astype(jnp.float32)
        tok_ref[pl.ds(i, 1)] = jnp.sum(
            mask1 * tokc + mask2 * tokc, axis=0,
            keepdims=True).reshape(1, 1, RBLK)
        w_ref[pl.ds(i, 1)] = jnp.sum(
            mask1 * w1 + mask2 * w2, axis=0,
            keepdims=True).reshape(1, 1, RBLK)
        return 0

    jax.lax.fori_loop(0, NPB, body, 0)

    endb = (pad_off + pcnt) * (1.0 / RBLK)
    eye8 = (e8i == e8j).astype(jnp.float32)
    endb_c = jax.lax.dot_general(eye8, endb, (((1,), (1,)), ((), ())),
                                 preferred_element_type=jnp.float32)
    ib = jax.lax.broadcasted_iota(jnp.int32, (1, NPB), 1).astype(jnp.float32)
    cnt = jnp.sum((endb_c <= ib).astype(jnp.float32), axis=0, keepdims=True)
    eblk = jnp.minimum(cnt, jnp.float32(N_EXP - 1))
    nact = jnp.sum(pcnt, axis=1, keepdims=True) * (1.0 / RBLK)
    nact = jnp.broadcast_to(nact, (1, NPB))
    s_ref[...] = jnp.concatenate([eblk, nact], axis=0).astype(jnp.int32)


def _moe_up_kernel(s_ref, tok_ref, g_ref, gw_ref, uw_ref, act_ref, *, T):
    i = pl.program_id(0)

    @pl.when(i < s_ref[1, 0])
    def _():
        tok = tok_ref[...].reshape(1, RBLK)
        ti = jax.lax.broadcasted_iota(jnp.int32,
                                      (T, RBLK), 0).astype(jnp.float32)
        G = (tok == ti).astype(jnp.bfloat16)
        xg = jax.lax.dot_general(G, g_ref[...], (((0,), (0,)), ((), ())),
                                 preferred_element_type=jnp.float32)
        xgb = xg.astype(jnp.bfloat16)
        h1 = jnp.dot(xgb, gw_ref[0], preferred_element_type=jnp.float32)
        h2 = jnp.dot(xgb, uw_ref[0], preferred_element_type=jnp.float32)
        act_ref[...] = (_silu(h1) * h2).astype(
            jnp.bfloat16).reshape(1, RBLK, INTER)


def _moe_down_kernel(s_ref, tok_ref, w_ref, act_ref, dw_ref, out_ref, *, T):
    i = pl.program_id(0)

    @pl.when(i == 0)
    def _():
        out_ref[...] = jnp.zeros_like(out_ref)

    @pl.when(i < s_ref[1, 0])
    def _():
        act = act_ref[...].reshape(RBLK, INTER)
        y = jnp.dot(act, dw_ref[0],
                    preferred_element_type=jnp.float32).astype(jnp.bfloat16)
        tok = tok_ref[...].reshape(1, RBLK)
        wrow = w_ref[...].reshape(1, RBLK)
        ti = jax.lax.broadcasted_iota(jnp.int32,
                                      (T, RBLK), 0).astype(jnp.float32)
        Gw = ((tok == ti).astype(jnp.float32)
              * wrow).astype(jnp.bfloat16)
        out_ref[...] += jnp.dot(Gw, y, preferred_element_type=jnp.float32)


def _final_kernel(h_ref, r_ref, g_ref, sgw_ref, suw_ref, sdw_ref, o_ref):
    g = g_ref[...]
    h1 = jnp.dot(g, sgw_ref[...], preferred_element_type=jnp.float32)
    h2 = jnp.dot(g, suw_ref[...], preferred_element_type=jnp.float32)
    act = (_silu(h1) * h2).astype(jnp.bfloat16)
    o_ref[...] = (h_ref[...] + r_ref[...]
                  + jnp.dot(act, sdw_ref[...],
                            preferred_element_type=jnp.float32))


def kernel(x, ln1_w, ln2_w, Wq, Wkva, Wkvb, Wo, gateW, expert_bias,
           gw, uw, dw, sgw, suw, sdw):
    B, T, C = x.shape
    xf = x.reshape(T, C)
    nt = T // TBLK
    NPB = (TOP_K * T) // RBLK + N_EXP

    freqs = 1.0 / (THETA ** (np.arange(0, ROPE_DIM, 2, dtype=np.float32)
                             / ROPE_DIM))
    f = np.outer(np.arange(T, dtype=np.float32), freqs)
    cosv = np.cos(f).astype(np.float32)
    sinv = np.sin(f).astype(np.float32)
    Ct = np.empty((T, ROPE_DIM), np.float32)
    St = np.empty((T, ROPE_DIM), np.float32)
    Ct[:, 0::2] = cosv
    Ct[:, 1::2] = cosv
    St[:, 0::2] = -sinv
    St[:, 1::2] = sinv
    Ct = jnp.asarray(Ct)
    St = jnp.asarray(St)

    qf, kf = pl.pallas_call(
        _pre_attn_kernel,
        grid=(nt,),
        in_specs=[
            pl.BlockSpec((TBLK, C), lambda i: (i, 0)),
            pl.BlockSpec((1, C), lambda i: (0, 0)),
            pl.BlockSpec((C, N_HEAD * HEAD_DIM), lambda i: (0, 0)),
            pl.BlockSpec((C, KV_LORA + ROPE_DIM), lambda i: (0, 0)),
            pl.BlockSpec((KV_LORA, N_HEAD * NOPE_DIM), lambda i: (0, 0)),
            pl.BlockSpec((TBLK, ROPE_DIM), lambda i: (i, 0)),
            pl.BlockSpec((TBLK, ROPE_DIM), lambda i: (i, 0)),
        ],
        out_specs=[pl.BlockSpec((N_HEAD, TBLK, HEAD_DIM), lambda i: (0, i, 0)),
                   pl.BlockSpec((N_HEAD, TBLK, HEAD_DIM), lambda i: (0, i, 0))],
        out_shape=[jax.ShapeDtypeStruct((N_HEAD, T, HEAD_DIM),
                                        jnp.float32)] * 2,
    )(xf, ln1_w.reshape(1, C), Wq, Wkva, Wkvb, Ct, St)

    yT = pl.pallas_call(
        functools.partial(_attn_kernel, T=T),
        grid=(N_HEAD, nt),
        in_specs=[
            pl.BlockSpec((1, TBLK, HEAD_DIM), lambda h, t: (h, t, 0)),
            pl.BlockSpec((1, T, HEAD_DIM), lambda h, t: (h, 0, 0)),
        ],
        out_specs=pl.BlockSpec((1, TBLK, HEAD_DIM), lambda h, t: (h, t, 0)),
        out_shape=jax.ShapeDtypeStruct((N_HEAD, T, HEAD_DIM), jnp.float32),
    )(qf, kf)
    y = yT.transpose(1, 0, 2).reshape(T, N_HEAD * HEAD_DIM)

    h, g, logits = pl.pallas_call(
        _post_attn_kernel,
        grid=(nt,),
        in_specs=[
            pl.BlockSpec((TBLK, N_HEAD * HEAD_DIM), lambda i: (i, 0)),
            pl.BlockSpec((N_HEAD * HEAD_DIM, C), lambda i: (0, 0)),
            pl.BlockSpec((TBLK, C), lambda i: (i, 0)),
            pl.BlockSpec((1, C), lambda i: (0, 0)),
            pl.BlockSpec((C, N_EXP), lambda i: (0, 0)),
            pl.BlockSpec((1, N_EXP), lambda i: (0, 0)),
        ],
        out_specs=[pl.BlockSpec((TBLK, C), lambda i: (i, 0)),
                   pl.BlockSpec((TBLK, C), lambda i: (i, 0)),
                   pl.BlockSpec((TBLK, N_EXP), lambda i: (i, 0))],
        out_shape=[jax.ShapeDtypeStruct((T, C), jnp.float32),
                   jax.ShapeDtypeStruct((T, C), jnp.bfloat16),
                   jax.ShapeDtypeStruct((T, N_EXP), jnp.float32)],
    )(y, Wo, xf, ln2_w.reshape(1, C), gateW, expert_bias.reshape(1, N_EXP))

    gwb = gw.astype(jnp.bfloat16)
    uwb = uw.astype(jnp.bfloat16)
    dwb = dw.astype(jnp.bfloat16)

    s, tokb, wb = pl.pallas_call(
        functools.partial(_route_kernel, T=T, NPB=NPB),
        in_specs=[pl.BlockSpec((T, N_EXP), lambda: (0, 0))],
        out_specs=[pl.BlockSpec((2, NPB), lambda: (0, 0)),
                   pl.BlockSpec((NPB, 1, RBLK), lambda: (0, 0, 0)),
                   pl.BlockSpec((NPB, 1, RBLK), lambda: (0, 0, 0))],
        out_shape=[jax.ShapeDtypeStruct((2, NPB), jnp.int32),
                   jax.ShapeDtypeStruct((NPB, 1, RBLK), jnp.float32),
                   jax.ShapeDtypeStruct((NPB, 1, RBLK), jnp.float32)],
    )(logits)

    act = pl.pallas_call(
        functools.partial(_moe_up_kernel, T=T),
        grid_spec=pltpu.PrefetchScalarGridSpec(
            num_scalar_prefetch=1,
            grid=(NPB,),
            in_specs=[
                pl.BlockSpec((1, 1, RBLK), lambda i, s: (i, 0, 0)),
                pl.BlockSpec((T, C), lambda i, s: (0, 0)),
                pl.BlockSpec((1, C, INTER), lambda i, s: (s[0, i], 0, 0)),
                pl.BlockSpec((1, C, INTER), lambda i, s: (s[0, i], 0, 0)),
            ],
            out_specs=pl.BlockSpec((1, RBLK, INTER), lambda i, s: (i, 0, 0)),
        ),
        out_shape=jax.ShapeDtypeStruct((NPB, RBLK, INTER), jnp.bfloat16),
    )(s, tokb, g, gwb, uwb)

    routed = pl.pallas_call(
        functools.partial(_moe_down_kernel, T=T),
        grid_spec=pltpu.PrefetchScalarGridSpec(
            num_scalar_prefetch=1,
            grid=(NPB,),
            in_specs=[
                pl.BlockSpec((1, 1, RBLK), lambda i, s: (i, 0, 0)),
                pl.BlockSpec((1, 1, RBLK), lambda i, s: (i, 0, 0)),
                pl.BlockSpec((1, RBLK, INTER), lambda i, s: (i, 0, 0)),
                pl.BlockSpec((1, INTER, C), lambda i, s: (s[0, i], 0, 0)),
            ],
            out_specs=pl.BlockSpec((T, C), lambda i, s: (0, 0)),
        ),
        out_shape=jax.ShapeDtypeStruct((T, C), jnp.float32),
    )(s, tokb, wb, act, dwb)

    out = pl.pallas_call(
        _final_kernel,
        grid=(nt,),
        in_specs=[
            pl.BlockSpec((TBLK, C), lambda i: (i, 0)),
            pl.BlockSpec((TBLK, C), lambda i: (i, 0)),
            pl.BlockSpec((TBLK, C), lambda i: (i, 0)),
            pl.BlockSpec((C, INTER), lambda i: (0, 0)),
            pl.BlockSpec((C, INTER), lambda i: (0, 0)),
            pl.BlockSpec((INTER, C), lambda i: (0, 0)),
        ],
        out_specs=pl.BlockSpec((TBLK, C), lambda i: (i, 0)),
        out_shape=jax.ShapeDtypeStruct((T, C), jnp.float32),
    )(h, routed, g, sgw.astype(jnp.bfloat16), suw.astype(jnp.bfloat16),
      sdw.astype(jnp.bfloat16))
    return out.reshape(B, T, C)


# speedup vs baseline: 1.1066x; 1.1066x over previous
import functools

import numpy as np
import jax
import jax.numpy as jnp
from jax.experimental import pallas as pl
from jax.experimental.pallas import tpu as pltpu

N_EMBD = 1024
N_HEAD = 16
HEAD_DIM = 64
KV_LORA = 256
ROPE_DIM = 32
NOPE_DIM = HEAD_DIM - ROPE_DIM
N_EXP = 8
TOP_K = 2
INTER = 2048
THETA = 100000.0

TBLK = 512
RBLK = 256
EPS = 1e-6


def _rms(x, w):
    return x * jax.lax.rsqrt(jnp.mean(x * x, axis=1, keepdims=True) + EPS) * w


def _silu(x):
    return x / (1.0 + jnp.exp(-x))


def _pre_attn_kernel(x_ref, w1_ref, wq_ref, wkva_ref, wkvb_ref, cos_ref,
                     sin_ref, q_ref, k_ref):
    g = _rms(x_ref[...], w1_ref[...])
    q = jnp.dot(g, wq_ref[...], preferred_element_type=jnp.float32)
    ckv = jnp.dot(g, wkva_ref[...], preferred_element_type=jnp.float32)
    latent = ckv[:, :KV_LORA]
    kr = ckv[:, KV_LORA:]
    kn = jnp.dot(latent, wkvb_ref[...], preferred_element_type=jnp.float32)
    C = cos_ref[...]
    S = sin_ref[...]
    di = jax.lax.broadcasted_iota(jnp.int32, (ROPE_DIM, ROPE_DIM), 0)
    dj = jax.lax.broadcasted_iota(jnp.int32, (ROPE_DIM, ROPE_DIM), 1)
    P = ((di ^ 1) == dj).astype(jnp.float32)

    def rope(v):
        return v * C + jnp.dot(v, P, preferred_element_type=jnp.float32) * S

    kr2 = rope(kr)
    qp = []
    kp = []
    for h in range(N_HEAD):
        qh = q[:, h * HEAD_DIM:(h + 1) * HEAD_DIM]
        qp.append(jnp.concatenate(
            [qh[:, :NOPE_DIM], rope(qh[:, NOPE_DIM:])],
            axis=1).reshape(1, TBLK, HEAD_DIM))
        kp.append(jnp.concatenate(
            [kn[:, h * NOPE_DIM:(h + 1) * NOPE_DIM], kr2],
            axis=1).reshape(1, TBLK, HEAD_DIM))
    q_ref[...] = jnp.concatenate(qp, axis=0)
    k_ref[...] = jnp.concatenate(kp, axis=0)


def _attn_kernel(q_ref, k_ref, o_ref, *, T):
    t = pl.program_id(1)
    q = q_ref[0]
    k = k_ref[0]
    s = jax.lax.dot_general(q, k, (((1,), (1,)), ((), ())),
                            preferred_element_type=jnp.float32)
    s = s * (1.0 / np.float32(np.sqrt(HEAD_DIM)))
    row = jax.lax.broadcasted_iota(jnp.int32, (TBLK, T), 0) + t * TBLK
    col = jax.lax.broadcasted_iota(jnp.int32, (TBLK, T), 1)
    s = jnp.where(col <= row, s, jnp.float32(-1e9))
    m = jnp.max(s, axis=1, keepdims=True)
    e = jnp.exp(s - m)
    p = e * (1.0 / jnp.sum(e, axis=1, keepdims=True))
    o_ref[0] = jnp.dot(p, k, preferred_element_type=jnp.float32)


def _post_attn_kernel(y_ref, wo_ref, x_ref, w2_ref, gate_ref, bias_ref,
                      h_ref, g_ref, logit_ref):
    h = x_ref[...] + jnp.dot(y_ref[...], wo_ref[...],
                             preferred_element_type=jnp.float32)
    h_ref[...] = h
    g = _rms(h, w2_ref[...])
    g_ref[...] = g
    logit_ref[...] = (jnp.dot(g, gate_ref[...],
                              preferred_element_type=jnp.float32)
                      + bias_ref[...])


def _route_kernel(logit_ref, s_ref, tok_ref, w_ref, *, T, NPB):
    lg = logit_ref[...]
    m = jnp.max(lg, axis=1, keepdims=True)
    ex = jnp.exp(lg - m)
    probs = ex / jnp.sum(ex, axis=1, keepdims=True)
    io8 = jax.lax.broadcasted_iota(jnp.int32, (T, N_EXP), 1).astype(jnp.float32)
    m1 = jnp.max(probs, axis=1, keepdims=True)
    a1 = jnp.min(jnp.where(probs == m1, io8, jnp.float32(N_EXP)),
                 axis=1, keepdims=True)
    oh1 = (io8 == a1).astype(jnp.float32)
    p2 = jnp.where(oh1 > 0, jnp.float32(-1.0), probs)
    m2 = jnp.max(p2, axis=1, keepdims=True)
    a2 = jnp.min(jnp.where(p2 == m2, io8, jnp.float32(N_EXP)),
                 axis=1, keepdims=True)
    oh2 = (io8 == a2).astype(jnp.float32)
    den = m1 + m2
    w1 = m1 / den
    w2 = m2 / den

    CH = 512
    ci = jax.lax.broadcasted_iota(jnp.int32, (CH, CH), 0)
    cj = jax.lax.broadcasted_iota(jnp.int32, (CH, CH), 1)
    tril = (cj < ci).astype(jnp.float32)

    def exclcum(oh):
        parts = []
        carry = jnp.zeros((1, N_EXP), jnp.float32)
        for c in range(T // CH):
            blk = oh[c * CH:(c + 1) * CH, :]
            parts.append(jnp.dot(tril, blk,
                                 preferred_element_type=jnp.float32) + carry)
            carry = carry + jnp.sum(blk, axis=0, keepdims=True)
        return jnp.concatenate(parts, axis=0), carry

    pos1, tot1 = exclcum(oh1)
    pos2, tot2 = exclcum(oh2)
    counts = tot1 + tot2
    pcnt = jnp.floor((counts + (RBLK - 1)) * (1.0 / RBLK)) * RBLK
    e8i = jax.lax.broadcasted_iota(jnp.int32, (N_EXP, N_EXP), 0)
    e8j = jax.lax.broadcasted_iota(jnp.int32, (N_EXP, N_EXP), 1)
    m8 = (e8i < e8j).astype(jnp.float32)
    pad_off = jnp.dot(pcnt, m8, preferred_element_type=jnp.float32)

    rank1 = jnp.sum(pos1 * oh1, axis=1, keepdims=True)
    rank2 = jnp.sum((pos2 + tot1) * oh2, axis=1, keepdims=True)
    dest1 = jnp.sum(pad_off * oh1, axis=1, keepdims=True) + rank1
    dest2 = jnp.sum(pad_off * oh2, axis=1, keepdims=True) + rank2

    tokc = jax.lax.broadcasted_iota(jnp.int32, (T, 1), 0).astype(jnp.float32)
    riota = jax.lax.broadcasted_iota(jnp.int32, (1, RBLK), 1)

    def body(i, _):
        r = (riota + i * RBLK).astype(jnp.float32)
        mask1 = (dest1 == r).astype(jnp.float32)
        mask2 = (dest2 == r).astype(jnp.float32)
        tok_ref[pl.ds(i, 1)] = jnp.sum(
            mask1 * tokc + mask2 * tokc, axis=0,
            keepdims=True).reshape(1, 1, RBLK)
        w_ref[pl.ds(i, 1)] = jnp.sum(
            mask1 * w1 + mask2 * w2, axis=0,
            keepdims=True).reshape(1, 1, RBLK)
        return 0

    jax.lax.fori_loop(0, NPB, body, 0)

    endb = (pad_off + pcnt) * (1.0 / RBLK)
    eye8 = (e8i == e8j).astype(jnp.float32)
    endb_c = jax.lax.dot_general(eye8, endb, (((1,), (1,)), ((), ())),
                                 preferred_element_type=jnp.float32)
    ib = jax.lax.broadcasted_iota(jnp.int32, (1, NPB), 1).astype(jnp.float32)
    cnt = jnp.sum((endb_c <= ib).astype(jnp.float32), axis=0, keepdims=True)
    eblk = jnp.minimum(cnt, jnp.float32(N_EXP - 1))
    nact = jnp.sum(pcnt, axis=1, keepdims=True) * (1.0 / RBLK)
    nact = jnp.broadcast_to(nact, (1, NPB))
    s_ref[...] = jnp.concatenate([eblk, nact], axis=0).astype(jnp.int32)


def _moe_up_kernel(s_ref, tok_ref, g_ref, gw_ref, uw_ref, act_ref, *, T):
    i = pl.program_id(0)

    @pl.when(i < s_ref[1, 0])
    def _():
        tok = tok_ref[...].reshape(1, RBLK)
        ti = jax.lax.broadcasted_iota(jnp.int32,
                                      (T, RBLK), 0).astype(jnp.float32)
        G = (tok == ti).astype(jnp.float32)
        xg = jax.lax.dot_general(G, g_ref[...], (((0,), (0,)), ((), ())),
                                 preferred_element_type=jnp.float32)
        h1 = jnp.dot(xg, gw_ref[0], preferred_element_type=jnp.float32)
        h2 = jnp.dot(xg, uw_ref[0], preferred_element_type=jnp.float32)
        act_ref[...] = (_silu(h1) * h2).reshape(1, RBLK, INTER)


def _moe_down_kernel(s_ref, tok_ref, w_ref, act_ref, dw_ref, out_ref, *, T):
    i = pl.program_id(0)

    @pl.when(i == 0)
    def _():
        out_ref[...] = jnp.zeros_like(out_ref)

    @pl.when(i < s_ref[1, 0])
    def _():
        act = act_ref[...].reshape(RBLK, INTER)
        y = jnp.dot(act, dw_ref[0], preferred_element_type=jnp.float32)
        tok = tok_ref[...].reshape(1, RBLK)
        wrow = w_ref[...].reshape(1, RBLK)
        ti = jax.lax.broadcasted_iota(jnp.int32,
                                      (T, RBLK), 0).astype(jnp.float32)
        Gw = (tok == ti).astype(jnp.float32) * wrow
        out_ref[...] += jnp.dot(Gw, y, preferred_element_type=jnp.float32)


def _final_kernel(h_ref, r_ref, g_ref, sgw_ref, suw_ref, sdw_ref, o_ref):
    g = g_ref[...]
    h1 = jnp.dot(g, sgw_ref[...], preferred_element_type=jnp.float32)
    h2 = jnp.dot(g, suw_ref[...], preferred_element_type=jnp.float32)
    act = _silu(h1) * h2
    o_ref[...] = (h_ref[...] + r_ref[...]
                  + jnp.dot(act, sdw_ref[...],
                            preferred_element_type=jnp.float32))


def kernel(x, ln1_w, ln2_w, Wq, Wkva, Wkvb, Wo, gateW, expert_bias,
           gw, uw, dw, sgw, suw, sdw):
    B, T, C = x.shape
    xf = x.reshape(T, C)
    nt = T // TBLK
    NPB = (TOP_K * T) // RBLK + N_EXP

    freqs = 1.0 / (THETA ** (np.arange(0, ROPE_DIM, 2, dtype=np.float32)
                             / ROPE_DIM))
    f = np.outer(np.arange(T, dtype=np.float32), freqs)
    cosv = np.cos(f).astype(np.float32)
    sinv = np.sin(f).astype(np.float32)
    Ct = np.empty((T, ROPE_DIM), np.float32)
    St = np.empty((T, ROPE_DIM), np.float32)
    Ct[:, 0::2] = cosv
    Ct[:, 1::2] = cosv
    St[:, 0::2] = -sinv
    St[:, 1::2] = sinv
    Ct = jnp.asarray(Ct)
    St = jnp.asarray(St)

    qf, kf = pl.pallas_call(
        _pre_attn_kernel,
        grid=(nt,),
        in_specs=[
            pl.BlockSpec((TBLK, C), lambda i: (i, 0)),
            pl.BlockSpec((1, C), lambda i: (0, 0)),
            pl.BlockSpec((C, N_HEAD * HEAD_DIM), lambda i: (0, 0)),
            pl.BlockSpec((C, KV_LORA + ROPE_DIM), lambda i: (0, 0)),
            pl.BlockSpec((KV_LORA, N_HEAD * NOPE_DIM), lambda i: (0, 0)),
            pl.BlockSpec((TBLK, ROPE_DIM), lambda i: (i, 0)),
            pl.BlockSpec((TBLK, ROPE_DIM), lambda i: (i, 0)),
        ],
        out_specs=[pl.BlockSpec((N_HEAD, TBLK, HEAD_DIM), lambda i: (0, i, 0)),
                   pl.BlockSpec((N_HEAD, TBLK, HEAD_DIM), lambda i: (0, i, 0))],
        out_shape=[jax.ShapeDtypeStruct((N_HEAD, T, HEAD_DIM),
                                        jnp.float32)] * 2,
    )(xf, ln1_w.reshape(1, C), Wq, Wkva, Wkvb, Ct, St)

    yT = pl.pallas_call(
        functools.partial(_attn_kernel, T=T),
        grid=(N_HEAD, nt),
        in_specs=[
            pl.BlockSpec((1, TBLK, HEAD_DIM), lambda h, t: (h, t, 0)),
            pl.BlockSpec((1, T, HEAD_DIM), lambda h, t: (h, 0, 0)),
        ],
        out_specs=pl.BlockSpec((1, TBLK, HEAD_DIM), lambda h, t: (h, t, 0)),
        out_shape=jax.ShapeDtypeStruct((N_HEAD, T, HEAD_DIM), jnp.float32),
    )(qf, kf)
    y = yT.transpose(1, 0, 2).reshape(T, N_HEAD * HEAD_DIM)

    h, g, logits = pl.pallas_call(
        _post_attn_kernel,
        grid=(nt,),
        in_specs=[
            pl.BlockSpec((TBLK, N_HEAD * HEAD_DIM), lambda i: (i, 0)),
            pl.BlockSpec((N_HEAD * HEAD_DIM, C), lambda i: (0, 0)),
            pl.BlockSpec((TBLK, C), lambda i: (i, 0)),
            pl.BlockSpec((1, C), lambda i: (0, 0)),
            pl.BlockSpec((C, N_EXP), lambda i: (0, 0)),
            pl.BlockSpec((1, N_EXP), lambda i: (0, 0)),
        ],
        out_specs=[pl.BlockSpec((TBLK, C), lambda i: (i, 0)),
                   pl.BlockSpec((TBLK, C), lambda i: (i, 0)),
                   pl.BlockSpec((TBLK, N_EXP), lambda i: (i, 0))],
        out_shape=[jax.ShapeDtypeStruct((T, C), jnp.float32),
                   jax.ShapeDtypeStruct((T, C), jnp.float32),
                   jax.ShapeDtypeStruct((T, N_EXP), jnp.float32)],
    )(y, Wo, xf, ln2_w.reshape(1, C), gateW, expert_bias.reshape(1, N_EXP))


    s, tokb, wb = pl.pallas_call(
        functools.partial(_route_kernel, T=T, NPB=NPB),
        in_specs=[pl.BlockSpec((T, N_EXP), lambda: (0, 0))],
        out_specs=[pl.BlockSpec((2, NPB), lambda: (0, 0)),
                   pl.BlockSpec((NPB, 1, RBLK), lambda: (0, 0, 0)),
                   pl.BlockSpec((NPB, 1, RBLK), lambda: (0, 0, 0))],
        out_shape=[jax.ShapeDtypeStruct((2, NPB), jnp.int32),
                   jax.ShapeDtypeStruct((NPB, 1, RBLK), jnp.float32),
                   jax.ShapeDtypeStruct((NPB, 1, RBLK), jnp.float32)],
    )(logits)

    act = pl.pallas_call(
        functools.partial(_moe_up_kernel, T=T),
        grid_spec=pltpu.PrefetchScalarGridSpec(
            num_scalar_prefetch=1,
            grid=(NPB,),
            in_specs=[
                pl.BlockSpec((1, 1, RBLK), lambda i, s: (i, 0, 0)),
                pl.BlockSpec((T, C), lambda i, s: (0, 0)),
                pl.BlockSpec((1, C, INTER), lambda i, s: (s[0, i], 0, 0)),
                pl.BlockSpec((1, C, INTER), lambda i, s: (s[0, i], 0, 0)),
            ],
            out_specs=pl.BlockSpec((1, RBLK, INTER), lambda i, s: (i, 0, 0)),
        ),
        out_shape=jax.ShapeDtypeStruct((NPB, RBLK, INTER), jnp.float32),
    )(s, tokb, g, gw, uw)

    routed = pl.pallas_call(
        functools.partial(_moe_down_kernel, T=T),
        grid_spec=pltpu.PrefetchScalarGridSpec(
            num_scalar_prefetch=1,
            grid=(NPB,),
            in_specs=[
                pl.BlockSpec((1, 1, RBLK), lambda i, s: (i, 0, 0)),
                pl.BlockSpec((1, 1, RBLK), lambda i, s: (i, 0, 0)),
                pl.BlockSpec((1, RBLK, INTER), lambda i, s: (i, 0, 0)),
                pl.BlockSpec((1, INTER, C), lambda i, s: (s[0, i], 0, 0)),
            ],
            out_specs=pl.BlockSpec((T, C), lambda i, s: (0, 0)),
        ),
        out_shape=jax.ShapeDtypeStruct((T, C), jnp.float32),
    )(s, tokb, wb, act, dw)

    out = pl.pallas_call(
        _final_kernel,
        grid=(nt,),
        in_specs=[
            pl.BlockSpec((TBLK, C), lambda i: (i, 0)),
            pl.BlockSpec((TBLK, C), lambda i: (i, 0)),
            pl.BlockSpec((TBLK, C), lambda i: (i, 0)),
            pl.BlockSpec((C, INTER), lambda i: (0, 0)),
            pl.BlockSpec((C, INTER), lambda i: (0, 0)),
            pl.BlockSpec((INTER, C), lambda i: (0, 0)),
        ],
        out_specs=pl.BlockSpec((TBLK, C), lambda i: (i, 0)),
        out_shape=jax.ShapeDtypeStruct((T, C), jnp.float32),
    )(h, routed, g, sgw, suw, sdw)
    return out.reshape(B, T, C)


# speedup vs baseline: 1.2661x; 1.1441x over previous
import functools

import numpy as np
import jax
import jax.numpy as jnp
from jax.experimental import pallas as pl
from jax.experimental.pallas import tpu as pltpu

N_EMBD = 1024
N_HEAD = 16
HEAD_DIM = 64
KV_LORA = 256
ROPE_DIM = 32
NOPE_DIM = HEAD_DIM - ROPE_DIM
N_EXP = 8
TOP_K = 2
INTER = 2048
THETA = 100000.0

TBLK = 512
RBLK = 256
EPS = 1e-6


def _rms(x, w):
    return x * jax.lax.rsqrt(jnp.mean(x * x, axis=1, keepdims=True) + EPS) * w


def _silu(x):
    return x / (1.0 + jnp.exp(-x))


def _pre_attn_kernel(x_ref, w1_ref, wq_ref, wkva_ref, wkvb_ref, cos_ref,
                     sin_ref, q_ref, k_ref):
    g = _rms(x_ref[...], w1_ref[...])
    q = jnp.dot(g, wq_ref[...], preferred_element_type=jnp.float32)
    ckv = jnp.dot(g, wkva_ref[...], preferred_element_type=jnp.float32)
    latent = ckv[:, :KV_LORA]
    kr = ckv[:, KV_LORA:]
    kn = jnp.dot(latent, wkvb_ref[...], preferred_element_type=jnp.float32)
    C = cos_ref[...]
    S = sin_ref[...]
    di = jax.lax.broadcasted_iota(jnp.int32, (ROPE_DIM, ROPE_DIM), 0)
    dj = jax.lax.broadcasted_iota(jnp.int32, (ROPE_DIM, ROPE_DIM), 1)
    P = ((di ^ 1) == dj).astype(jnp.float32)

    def rope(v):
        return v * C + jnp.dot(v, P, preferred_element_type=jnp.float32) * S

    kr2 = rope(kr)
    qp = []
    kp = []
    for h in range(N_HEAD):
        qh = q[:, h * HEAD_DIM:(h + 1) * HEAD_DIM]
        qp.append(jnp.concatenate(
            [qh[:, :NOPE_DIM], rope(qh[:, NOPE_DIM:])],
            axis=1).reshape(1, TBLK, HEAD_DIM))
        kp.append(jnp.concatenate(
            [kn[:, h * NOPE_DIM:(h + 1) * NOPE_DIM], kr2],
            axis=1).reshape(1, TBLK, HEAD_DIM))
    q_ref[...] = jnp.concatenate(qp, axis=0)
    k_ref[...] = jnp.concatenate(kp, axis=0)


def _attn_kernel(q_ref, k_ref, o_ref, *, T):
    t = pl.program_id(1)
    q = q_ref[0]
    scale = 1.0 / np.float32(np.sqrt(HEAD_DIM))
    rowi = jax.lax.broadcasted_iota(jnp.int32, (TBLK, TBLK), 0) + t * TBLK
    coli = jax.lax.broadcasted_iota(jnp.int32, (TBLK, TBLK), 1)

    def body(j, carry):
        m0, l0, acc0 = carry
        ks = k_ref[0, pl.ds(j * TBLK, TBLK), :]
        s = jax.lax.dot_general(q, ks, (((1,), (1,)), ((), ())),
                                preferred_element_type=jnp.float32) * scale
        s = jnp.where(coli + j * TBLK <= rowi, s, jnp.float32(-1e9))
        m1 = jnp.maximum(m0, jnp.max(s, axis=1, keepdims=True))
        alpha = jnp.exp(m0 - m1)
        p = jnp.exp(s - m1)
        l1 = l0 * alpha + jnp.sum(p, axis=1, keepdims=True)
        acc1 = acc0 * alpha + jnp.dot(p, ks,
                                      preferred_element_type=jnp.float32)
        return m1, l1, acc1

    m0 = jnp.full((TBLK, 1), jnp.float32(-1e30))
    l0 = jnp.zeros((TBLK, 1), jnp.float32)
    acc0 = jnp.zeros((TBLK, HEAD_DIM), jnp.float32)
    m, l, acc = jax.lax.fori_loop(0, t + 1, body, (m0, l0, acc0))
    o_ref[0] = acc * (1.0 / l)


def _post_attn_kernel(y_ref, wo_ref, x_ref, w2_ref, gate_ref, bias_ref,
                      h_ref, g_ref, logit_ref):
    h = x_ref[...] + jnp.dot(y_ref[...], wo_ref[...],
                             preferred_element_type=jnp.float32)
    h_ref[...] = h
    g = _rms(h, w2_ref[...])
    g_ref[...] = g
    logit_ref[...] = (jnp.dot(g, gate_ref[...],
                              preferred_element_type=jnp.float32)
                      + bias_ref[...])


def _route_kernel(logit_ref, s_ref, tok_ref, w_ref, *, T, NPB):
    lg = logit_ref[...]
    m = jnp.max(lg, axis=1, keepdims=True)
    ex = jnp.exp(lg - m)
    probs = ex / jnp.sum(ex, axis=1, keepdims=True)
    io8 = jax.lax.broadcasted_iota(jnp.int32, (T, N_EXP), 1).astype(jnp.float32)
    m1 = jnp.max(probs, axis=1, keepdims=True)
    a1 = jnp.min(jnp.where(probs == m1, io8, jnp.float32(N_EXP)),
                 axis=1, keepdims=True)
    oh1 = (io8 == a1).astype(jnp.float32)
    p2 = jnp.where(oh1 > 0, jnp.float32(-1.0), probs)
    m2 = jnp.max(p2, axis=1, keepdims=True)
    a2 = jnp.min(jnp.where(p2 == m2, io8, jnp.float32(N_EXP)),
                 axis=1, keepdims=True)
    oh2 = (io8 == a2).astype(jnp.float32)
    den = m1 + m2
    w1 = m1 / den
    w2 = m2 / den

    CH = 512
    ci = jax.lax.broadcasted_iota(jnp.int32, (CH, CH), 0)
    cj = jax.lax.broadcasted_iota(jnp.int32, (CH, CH), 1)
    tril = (cj < ci).astype(jnp.float32)

    def exclcum(oh):
        parts = []
        carry = jnp.zeros((1, N_EXP), jnp.float32)
        for c in range(T // CH):
            blk = oh[c * CH:(c + 1) * CH, :]
            parts.append(jnp.dot(tril, blk,
                                 preferred_element_type=jnp.float32) + carry)
            carry = carry + jnp.sum(blk, axis=0, keepdims=True)
        return jnp.concatenate(parts, axis=0), carry

    pos1, tot1 = exclcum(oh1)
    pos2, tot2 = exclcum(oh2)
    counts = tot1 + tot2
    pcnt = jnp.floor((counts + (RBLK - 1)) * (1.0 / RBLK)) * RBLK
    e8i = jax.lax.broadcasted_iota(jnp.int32, (N_EXP, N_EXP), 0)
    e8j = jax.lax.broadcasted_iota(jnp.int32, (N_EXP, N_EXP), 1)
    m8 = (e8i < e8j).astype(jnp.float32)
    pad_off = jnp.dot(pcnt, m8, preferred_element_type=jnp.float32)

    rank1 = jnp.sum(pos1 * oh1, axis=1, keepdims=True)
    rank2 = jnp.sum((pos2 + tot1) * oh2, axis=1, keepdims=True)
    dest1 = jnp.sum(pad_off * oh1, axis=1, keepdims=True) + rank1
    dest2 = jnp.sum(pad_off * oh2, axis=1, keepdims=True) + rank2

    tokc = jax.lax.broadcasted_iota(jnp.int32, (T, 1), 0).astype(jnp.float32)
    riota = jax.lax.broadcasted_iota(jnp.int32, (1, RBLK), 1)

    def body(i, _):
        r = (riota + i * RBLK).astype(jnp.float32)
        mask1 = (dest1 == r).astype(jnp.float32)
        mask2 = (dest2 == r).astype(jnp.float32)
        tok_ref[pl.ds(i, 1)] = jnp.sum(
            mask1 * tokc + mask2 * tokc, axis=0,
            keepdims=True).reshape(1, 1, RBLK)
        w_ref[pl.ds(i, 1)] = jnp.sum(
            mask1 * w1 + mask2 * w2, axis=0,
            keepdims=True).reshape(1, 1, RBLK)
        return 0

    jax.lax.fori_loop(0, NPB, body, 0)

    endb = (pad_off + pcnt) * (1.0 / RBLK)
    eye8 = (e8i == e8j).astype(jnp.float32)
    endb_c = jax.lax.dot_general(eye8, endb, (((1,), (1,)), ((), ())),
                                 preferred_element_type=jnp.float32)
    ib = jax.lax.broadcasted_iota(jnp.int32, (1, NPB), 1).astype(jnp.float32)
    cnt = jnp.sum((endb_c <= ib).astype(jnp.float32), axis=0, keepdims=True)
    eblk = jnp.minimum(cnt, jnp.float32(N_EXP - 1))
    nact = jnp.sum(pcnt, axis=1, keepdims=True) * (1.0 / RBLK)
    nact = jnp.broadcast_to(nact, (1, NPB))
    s_ref[...] = jnp.concatenate([eblk, nact], axis=0).astype(jnp.int32)


def _moe_up_kernel(s_ref, tok_ref, g_ref, gw_ref, uw_ref, act_ref, *, T):
    i = pl.program_id(0)

    @pl.when(i < s_ref[1, 0])
    def _():
        tok = tok_ref[...].reshape(1, RBLK)
        ti = jax.lax.broadcasted_iota(jnp.int32,
                                      (T, RBLK), 0).astype(jnp.float32)
        G = (tok == ti).astype(jnp.float32)
        xg = jax.lax.dot_general(G, g_ref[...], (((0,), (0,)), ((), ())),
                                 preferred_element_type=jnp.float32)
        h1 = jnp.dot(xg, gw_ref[0], preferred_element_type=jnp.float32)
        h2 = jnp.dot(xg, uw_ref[0], preferred_element_type=jnp.float32)
        act_ref[...] = (_silu(h1) * h2).reshape(1, RBLK, INTER)


def _moe_down_kernel(s_ref, tok_ref, w_ref, act_ref, dw_ref, out_ref, *, T):
    i = pl.program_id(0)

    @pl.when(i == 0)
    def _():
        out_ref[...] = jnp.zeros_like(out_ref)

    @pl.when(i < s_ref[1, 0])
    def _():
        act = act_ref[...].reshape(RBLK, INTER)
        y = jnp.dot(act, dw_ref[0], preferred_element_type=jnp.float32)
        tok = tok_ref[...].reshape(1, RBLK)
        wrow = w_ref[...].reshape(1, RBLK)
        ti = jax.lax.broadcasted_iota(jnp.int32,
                                      (T, RBLK), 0).astype(jnp.float32)
        Gw = (tok == ti).astype(jnp.float32) * wrow
        out_ref[...] += jnp.dot(Gw, y, preferred_element_type=jnp.float32)


def _final_kernel(h_ref, r_ref, g_ref, sgw_ref, suw_ref, sdw_ref, o_ref):
    g = g_ref[...]
    h1 = jnp.dot(g, sgw_ref[...], preferred_element_type=jnp.float32)
    h2 = jnp.dot(g, suw_ref[...], preferred_element_type=jnp.float32)
    act = _silu(h1) * h2
    o_ref[...] = (h_ref[...] + r_ref[...]
                  + jnp.dot(act, sdw_ref[...],
                            preferred_element_type=jnp.float32))


def kernel(x, ln1_w, ln2_w, Wq, Wkva, Wkvb, Wo, gateW, expert_bias,
           gw, uw, dw, sgw, suw, sdw):
    B, T, C = x.shape
    xf = x.reshape(T, C)
    nt = T // TBLK
    NPB = (TOP_K * T) // RBLK + N_EXP

    freqs = 1.0 / (THETA ** (np.arange(0, ROPE_DIM, 2, dtype=np.float32)
                             / ROPE_DIM))
    f = np.outer(np.arange(T, dtype=np.float32), freqs)
    cosv = np.cos(f).astype(np.float32)
    sinv = np.sin(f).astype(np.float32)
    Ct = np.empty((T, ROPE_DIM), np.float32)
    St = np.empty((T, ROPE_DIM), np.float32)
    Ct[:, 0::2] = cosv
    Ct[:, 1::2] = cosv
    St[:, 0::2] = -sinv
    St[:, 1::2] = sinv
    Ct = jnp.asarray(Ct)
    St = jnp.asarray(St)

    qf, kf = pl.pallas_call(
        _pre_attn_kernel,
        grid=(nt,),
        in_specs=[
            pl.BlockSpec((TBLK, C), lambda i: (i, 0)),
            pl.BlockSpec((1, C), lambda i: (0, 0)),
            pl.BlockSpec((C, N_HEAD * HEAD_DIM), lambda i: (0, 0)),
            pl.BlockSpec((C, KV_LORA + ROPE_DIM), lambda i: (0, 0)),
            pl.BlockSpec((KV_LORA, N_HEAD * NOPE_DIM), lambda i: (0, 0)),
            pl.BlockSpec((TBLK, ROPE_DIM), lambda i: (i, 0)),
            pl.BlockSpec((TBLK, ROPE_DIM), lambda i: (i, 0)),
        ],
        out_specs=[pl.BlockSpec((N_HEAD, TBLK, HEAD_DIM), lambda i: (0, i, 0)),
                   pl.BlockSpec((N_HEAD, TBLK, HEAD_DIM), lambda i: (0, i, 0))],
        out_shape=[jax.ShapeDtypeStruct((N_HEAD, T, HEAD_DIM),
                                        jnp.float32)] * 2,
    )(xf, ln1_w.reshape(1, C), Wq, Wkva, Wkvb, Ct, St)

    yT = pl.pallas_call(
        functools.partial(_attn_kernel, T=T),
        grid=(N_HEAD, nt),
        in_specs=[
            pl.BlockSpec((1, TBLK, HEAD_DIM), lambda h, t: (h, t, 0)),
            pl.BlockSpec((1, T, HEAD_DIM), lambda h, t: (h, 0, 0)),
        ],
        out_specs=pl.BlockSpec((1, TBLK, HEAD_DIM), lambda h, t: (h, t, 0)),
        out_shape=jax.ShapeDtypeStruct((N_HEAD, T, HEAD_DIM), jnp.float32),
    )(qf, kf)
    y = yT.transpose(1, 0, 2).reshape(T, N_HEAD * HEAD_DIM)

    h, g, logits = pl.pallas_call(
        _post_attn_kernel,
        grid=(nt,),
        in_specs=[
            pl.BlockSpec((TBLK, N_HEAD * HEAD_DIM), lambda i: (i, 0)),
            pl.BlockSpec((N_HEAD * HEAD_DIM, C), lambda i: (0, 0)),
            pl.BlockSpec((TBLK, C), lambda i: (i, 0)),
            pl.BlockSpec((1, C), lambda i: (0, 0)),
            pl.BlockSpec((C, N_EXP), lambda i: (0, 0)),
            pl.BlockSpec((1, N_EXP), lambda i: (0, 0)),
        ],
        out_specs=[pl.BlockSpec((TBLK, C), lambda i: (i, 0)),
                   pl.BlockSpec((TBLK, C), lambda i: (i, 0)),
                   pl.BlockSpec((TBLK, N_EXP), lambda i: (i, 0))],
        out_shape=[jax.ShapeDtypeStruct((T, C), jnp.float32),
                   jax.ShapeDtypeStruct((T, C), jnp.float32),
                   jax.ShapeDtypeStruct((T, N_EXP), jnp.float32)],
    )(y, Wo, xf, ln2_w.reshape(1, C), gateW, expert_bias.reshape(1, N_EXP))


    s, tokb, wb = pl.pallas_call(
        functools.partial(_route_kernel, T=T, NPB=NPB),
        in_specs=[pl.BlockSpec((T, N_EXP), lambda: (0, 0))],
        out_specs=[pl.BlockSpec((2, NPB), lambda: (0, 0)),
                   pl.BlockSpec((NPB, 1, RBLK), lambda: (0, 0, 0)),
                   pl.BlockSpec((NPB, 1, RBLK), lambda: (0, 0, 0))],
        out_shape=[jax.ShapeDtypeStruct((2, NPB), jnp.int32),
                   jax.ShapeDtypeStruct((NPB, 1, RBLK), jnp.float32),
                   jax.ShapeDtypeStruct((NPB, 1, RBLK), jnp.float32)],
    )(logits)

    act = pl.pallas_call(
        functools.partial(_moe_up_kernel, T=T),
        grid_spec=pltpu.PrefetchScalarGridSpec(
            num_scalar_prefetch=1,
            grid=(NPB,),
            in_specs=[
                pl.BlockSpec((1, 1, RBLK), lambda i, s: (i, 0, 0)),
                pl.BlockSpec((T, C), lambda i, s: (0, 0)),
                pl.BlockSpec((1, C, INTER), lambda i, s: (s[0, i], 0, 0)),
                pl.BlockSpec((1, C, INTER), lambda i, s: (s[0, i], 0, 0)),
            ],
            out_specs=pl.BlockSpec((1, RBLK, INTER), lambda i, s: (i, 0, 0)),
        ),
        out_shape=jax.ShapeDtypeStruct((NPB, RBLK, INTER), jnp.float32),
    )(s, tokb, g, gw, uw)

    routed = pl.pallas_call(
        functools.partial(_moe_down_kernel, T=T),
        grid_spec=pltpu.PrefetchScalarGridSpec(
            num_scalar_prefetch=1,
            grid=(NPB,),
            in_specs=[
                pl.BlockSpec((1, 1, RBLK), lambda i, s: (i, 0, 0)),
                pl.BlockSpec((1, 1, RBLK), lambda i, s: (i, 0, 0)),
                pl.BlockSpec((1, RBLK, INTER), lambda i, s: (i, 0, 0)),
                pl.BlockSpec((1, INTER, C), lambda i, s: (s[0, i], 0, 0)),
            ],
            out_specs=pl.BlockSpec((T, C), lambda i, s: (0, 0)),
        ),
        out_shape=jax.ShapeDtypeStruct((T, C), jnp.float32),
    )(s, tokb, wb, act, dw)

    out = pl.pallas_call(
        _final_kernel,
        grid=(nt,),
        in_specs=[
            pl.BlockSpec((TBLK, C), lambda i: (i, 0)),
            pl.BlockSpec((TBLK, C), lambda i: (i, 0)),
            pl.BlockSpec((TBLK, C), lambda i: (i, 0)),
            pl.BlockSpec((C, INTER), lambda i: (0, 0)),
            pl.BlockSpec((C, INTER), lambda i: (0, 0)),
            pl.BlockSpec((INTER, C), lambda i: (0, 0)),
        ],
        out_specs=pl.BlockSpec((TBLK, C), lambda i: (i, 0)),
        out_shape=jax.ShapeDtypeStruct((T, C), jnp.float32),
    )(h, routed, g, sgw, suw, sdw)
    return out.reshape(B, T, C)


# speedup vs baseline: 1.2682x; 1.0017x over previous
import functools

import numpy as np
import jax
import jax.numpy as jnp
from jax.experimental import pallas as pl
from jax.experimental.pallas import tpu as pltpu

N_EMBD = 1024
N_HEAD = 16
HEAD_DIM = 64
KV_LORA = 256
ROPE_DIM = 32
NOPE_DIM = HEAD_DIM - ROPE_DIM
N_EXP = 8
TOP_K = 2
INTER = 2048
THETA = 100000.0

TBLK = 512
RBLK = 256
EPS = 1e-6


def _rms(x, w):
    return x * jax.lax.rsqrt(jnp.mean(x * x, axis=1, keepdims=True) + EPS) * w


def _silu(x):
    return x / (1.0 + jnp.exp(-x))


def _pre_attn_kernel(x_ref, w1_ref, wq_ref, wkva_ref, wkvb_ref, cos_ref,
                     sin_ref, q_ref, k_ref):
    g = _rms(x_ref[...], w1_ref[...])
    q = jnp.dot(g, wq_ref[...], preferred_element_type=jnp.float32)
    ckv = jnp.dot(g, wkva_ref[...], preferred_element_type=jnp.float32)
    latent = ckv[:, :KV_LORA]
    kr = ckv[:, KV_LORA:]
    kn = jnp.dot(latent, wkvb_ref[...], preferred_element_type=jnp.float32)
    C = cos_ref[...]
    S = sin_ref[...]
    di = jax.lax.broadcasted_iota(jnp.int32, (ROPE_DIM, ROPE_DIM), 0)
    dj = jax.lax.broadcasted_iota(jnp.int32, (ROPE_DIM, ROPE_DIM), 1)
    P = ((di ^ 1) == dj).astype(jnp.float32)

    def rope(v):
        return v * C + jnp.dot(v, P, preferred_element_type=jnp.float32) * S

    kr2 = rope(kr)
    qp = []
    kp = []
    for h in range(N_HEAD):
        qh = q[:, h * HEAD_DIM:(h + 1) * HEAD_DIM]
        qp.append(jnp.concatenate(
            [qh[:, :NOPE_DIM], rope(qh[:, NOPE_DIM:])],
            axis=1).reshape(1, TBLK, HEAD_DIM))
        kp.append(jnp.concatenate(
            [kn[:, h * NOPE_DIM:(h + 1) * NOPE_DIM], kr2],
            axis=1).reshape(1, TBLK, HEAD_DIM))
    q_ref[...] = jnp.concatenate(qp, axis=0)
    k_ref[...] = jnp.concatenate(kp, axis=0)


def _attn_kernel(q_ref, k_ref, o_ref, *, T):
    t = pl.program_id(1)
    q = q_ref[0]
    scale = 1.0 / np.float32(np.sqrt(HEAD_DIM))
    rowi = jax.lax.broadcasted_iota(jnp.int32, (TBLK, TBLK), 0) + t * TBLK
    coli = jax.lax.broadcasted_iota(jnp.int32, (TBLK, TBLK), 1)

    def body(j, carry):
        m0, l0, acc0 = carry
        ks = k_ref[0, pl.ds(j * TBLK, TBLK), :]
        s = jax.lax.dot_general(q, ks, (((1,), (1,)), ((), ())),
                                preferred_element_type=jnp.float32) * scale
        s = jnp.where(coli + j * TBLK <= rowi, s, jnp.float32(-1e9))
        m1 = jnp.maximum(m0, jnp.max(s, axis=1, keepdims=True))
        alpha = jnp.exp(m0 - m1)
        p = jnp.exp(s - m1)
        l1 = l0 * alpha + jnp.sum(p, axis=1, keepdims=True)
        acc1 = acc0 * alpha + jnp.dot(p, ks,
                                      preferred_element_type=jnp.float32)
        return m1, l1, acc1

    m0 = jnp.full((TBLK, 1), jnp.float32(-1e30))
    l0 = jnp.zeros((TBLK, 1), jnp.float32)
    acc0 = jnp.zeros((TBLK, HEAD_DIM), jnp.float32)
    m, l, acc = jax.lax.fori_loop(0, t + 1, body, (m0, l0, acc0))
    o_ref[0] = acc * (1.0 / l)


def _post_attn_kernel(y_ref, wo_ref, x_ref, w2_ref, gate_ref, bias_ref,
                      h_ref, g_ref, logit_ref):
    h = x_ref[...] + jnp.dot(y_ref[...], wo_ref[...],
                             preferred_element_type=jnp.float32)
    h_ref[...] = h
    g = _rms(h, w2_ref[...])
    g_ref[...] = g
    logit_ref[...] = (jnp.dot(g, gate_ref[...],
                              preferred_element_type=jnp.float32)
                      + bias_ref[...])


def _route_kernel(logit_ref, s_ref, tok_ref, w_ref, *, T, NPB):
    lg = logit_ref[...]
    m = jnp.max(lg, axis=1, keepdims=True)
    ex = jnp.exp(lg - m)
    probs = ex / jnp.sum(ex, axis=1, keepdims=True)
    io8 = jax.lax.broadcasted_iota(jnp.int32, (T, N_EXP), 1).astype(jnp.float32)
    m1 = jnp.max(probs, axis=1, keepdims=True)
    a1 = jnp.min(jnp.where(probs == m1, io8, jnp.float32(N_EXP)),
                 axis=1, keepdims=True)
    oh1 = (io8 == a1).astype(jnp.float32)
    p2 = jnp.where(oh1 > 0, jnp.float32(-1.0), probs)
    m2 = jnp.max(p2, axis=1, keepdims=True)
    a2 = jnp.min(jnp.where(p2 == m2, io8, jnp.float32(N_EXP)),
                 axis=1, keepdims=True)
    oh2 = (io8 == a2).astype(jnp.float32)
    den = m1 + m2
    w1 = m1 / den
    w2 = m2 / den

    CH = 512
    ci = jax.lax.broadcasted_iota(jnp.int32, (CH, CH), 0)
    cj = jax.lax.broadcasted_iota(jnp.int32, (CH, CH), 1)
    tril = (cj < ci).astype(jnp.float32)

    def exclcum(oh):
        parts = []
        carry = jnp.zeros((1, N_EXP), jnp.float32)
        for c in range(T // CH):
            blk = oh[c * CH:(c + 1) * CH, :]
            parts.append(jnp.dot(tril, blk,
                                 preferred_element_type=jnp.float32) + carry)
            carry = carry + jnp.sum(blk, axis=0, keepdims=True)
        return jnp.concatenate(parts, axis=0), carry

    pos1, tot1 = exclcum(oh1)
    pos2, tot2 = exclcum(oh2)
    counts = tot1 + tot2
    pcnt = jnp.floor((counts + (RBLK - 1)) * (1.0 / RBLK)) * RBLK
    e8i = jax.lax.broadcasted_iota(jnp.int32, (N_EXP, N_EXP), 0)
    e8j = jax.lax.broadcasted_iota(jnp.int32, (N_EXP, N_EXP), 1)
    m8 = (e8i < e8j).astype(jnp.float32)
    pad_off = jnp.dot(pcnt, m8, preferred_element_type=jnp.float32)

    rank1 = jnp.sum(pos1 * oh1, axis=1, keepdims=True)
    rank2 = jnp.sum((pos2 + tot1) * oh2, axis=1, keepdims=True)
    dest1 = jnp.sum(pad_off * oh1, axis=1, keepdims=True) + rank1
    dest2 = jnp.sum(pad_off * oh2, axis=1, keepdims=True) + rank2

    tokc = jax.lax.broadcasted_iota(jnp.int32, (T, 1), 0).astype(jnp.float32)
    riota = jax.lax.broadcasted_iota(jnp.int32, (1, RBLK), 1)

    c0 = (((0,), (0,)), ((), ()))

    def body(i, _):
        r = (riota + i * RBLK).astype(jnp.float32)
        mask1 = (dest1 == r).astype(jnp.float32)
        mask2 = (dest2 == r).astype(jnp.float32)
        tok_ref[pl.ds(i, 1)] = (
            jax.lax.dot_general(tokc, mask1, c0,
                                preferred_element_type=jnp.float32)
            + jax.lax.dot_general(tokc, mask2, c0,
                                  preferred_element_type=jnp.float32)
        ).reshape(1, 1, RBLK)
        w_ref[pl.ds(i, 1)] = (
            jax.lax.dot_general(w1, mask1, c0,
                                preferred_element_type=jnp.float32)
            + jax.lax.dot_general(w2, mask2, c0,
                                  preferred_element_type=jnp.float32)
        ).reshape(1, 1, RBLK)
        return 0

    jax.lax.fori_loop(0, NPB, body, 0)

    endb = (pad_off + pcnt) * (1.0 / RBLK)
    eye8 = (e8i == e8j).astype(jnp.float32)
    endb_c = jax.lax.dot_general(eye8, endb, (((1,), (1,)), ((), ())),
                                 preferred_element_type=jnp.float32)
    ib = jax.lax.broadcasted_iota(jnp.int32, (1, NPB), 1).astype(jnp.float32)
    cnt = jnp.sum((endb_c <= ib).astype(jnp.float32), axis=0, keepdims=True)
    eblk = jnp.minimum(cnt, jnp.float32(N_EXP - 1))
    nact = jnp.sum(pcnt, axis=1, keepdims=True) * (1.0 / RBLK)
    nact = jnp.broadcast_to(nact, (1, NPB))
    s_ref[...] = jnp.concatenate([eblk, nact], axis=0).astype(jnp.int32)


def _moe_up_kernel(s_ref, tok_ref, g_ref, gw_ref, uw_ref, act_ref, *, T):
    i = pl.program_id(0)

    @pl.when(i < s_ref[1, 0])
    def _():
        tok = tok_ref[...].reshape(1, RBLK)
        ti = jax.lax.broadcasted_iota(jnp.int32,
                                      (T, RBLK), 0).astype(jnp.float32)
        G = (tok == ti).astype(jnp.float32)
        xg = jax.lax.dot_general(G, g_ref[...], (((0,), (0,)), ((), ())),
                                 preferred_element_type=jnp.float32)
        h1 = jnp.dot(xg, gw_ref[0], preferred_element_type=jnp.float32)
        h2 = jnp.dot(xg, uw_ref[0], preferred_element_type=jnp.float32)
        act_ref[...] = (_silu(h1) * h2).reshape(1, RBLK, INTER)


def _moe_down_kernel(s_ref, tok_ref, w_ref, act_ref, dw_ref, out_ref, *, T):
    i = pl.program_id(0)

    @pl.when(i == 0)
    def _():
        out_ref[...] = jnp.zeros_like(out_ref)

    @pl.when(i < s_ref[1, 0])
    def _():
        act = act_ref[...].reshape(RBLK, INTER)
        y = jnp.dot(act, dw_ref[0], preferred_element_type=jnp.float32)
        tok = tok_ref[...].reshape(1, RBLK)
        wrow = w_ref[...].reshape(1, RBLK)
        ti = jax.lax.broadcasted_iota(jnp.int32,
                                      (T, RBLK), 0).astype(jnp.float32)
        Gw = (tok == ti).astype(jnp.float32) * wrow
        out_ref[...] += jnp.dot(Gw, y, preferred_element_type=jnp.float32)


def _final_kernel(h_ref, r_ref, g_ref, sgw_ref, suw_ref, sdw_ref, o_ref):
    g = g_ref[...]
    h1 = jnp.dot(g, sgw_ref[...], preferred_element_type=jnp.float32)
    h2 = jnp.dot(g, suw_ref[...], preferred_element_type=jnp.float32)
    act = _silu(h1) * h2
    o_ref[...] = (h_ref[...] + r_ref[...]
                  + jnp.dot(act, sdw_ref[...],
                            preferred_element_type=jnp.float32))


def kernel(x, ln1_w, ln2_w, Wq, Wkva, Wkvb, Wo, gateW, expert_bias,
           gw, uw, dw, sgw, suw, sdw):
    B, T, C = x.shape
    xf = x.reshape(T, C)
    nt = T // TBLK
    NPB = (TOP_K * T) // RBLK + N_EXP

    freqs = 1.0 / (THETA ** (np.arange(0, ROPE_DIM, 2, dtype=np.float32)
                             / ROPE_DIM))
    f = np.outer(np.arange(T, dtype=np.float32), freqs)
    cosv = np.cos(f).astype(np.float32)
    sinv = np.sin(f).astype(np.float32)
    Ct = np.empty((T, ROPE_DIM), np.float32)
    St = np.empty((T, ROPE_DIM), np.float32)
    Ct[:, 0::2] = cosv
    Ct[:, 1::2] = cosv
    St[:, 0::2] = -sinv
    St[:, 1::2] = sinv
    Ct = jnp.asarray(Ct)
    St = jnp.asarray(St)

    qf, kf = pl.pallas_call(
        _pre_attn_kernel,
        grid=(nt,),
        in_specs=[
            pl.BlockSpec((TBLK, C), lambda i: (i, 0)),
            pl.BlockSpec((1, C), lambda i: (0, 0)),
            pl.BlockSpec((C, N_HEAD * HEAD_DIM), lambda i: (0, 0)),
            pl.BlockSpec((C, KV_LORA + ROPE_DIM), lambda i: (0, 0)),
            pl.BlockSpec((KV_LORA, N_HEAD * NOPE_DIM), lambda i: (0, 0)),
            pl.BlockSpec((TBLK, ROPE_DIM), lambda i: (i, 0)),
            pl.BlockSpec((TBLK, ROPE_DIM), lambda i: (i, 0)),
        ],
        out_specs=[pl.BlockSpec((N_HEAD, TBLK, HEAD_DIM), lambda i: (0, i, 0)),
                   pl.BlockSpec((N_HEAD, TBLK, HEAD_DIM), lambda i: (0, i, 0))],
        out_shape=[jax.ShapeDtypeStruct((N_HEAD, T, HEAD_DIM),
                                        jnp.float32)] * 2,
    )(xf, ln1_w.reshape(1, C), Wq, Wkva, Wkvb, Ct, St)

    yT = pl.pallas_call(
        functools.partial(_attn_kernel, T=T),
        grid=(N_HEAD, nt),
        in_specs=[
            pl.BlockSpec((1, TBLK, HEAD_DIM), lambda h, t: (h, t, 0)),
            pl.BlockSpec((1, T, HEAD_DIM), lambda h, t: (h, 0, 0)),
        ],
        out_specs=pl.BlockSpec((1, TBLK, HEAD_DIM), lambda h, t: (h, t, 0)),
        out_shape=jax.ShapeDtypeStruct((N_HEAD, T, HEAD_DIM), jnp.float32),
    )(qf, kf)
    y = yT.transpose(1, 0, 2).reshape(T, N_HEAD * HEAD_DIM)

    h, g, logits = pl.pallas_call(
        _post_attn_kernel,
        grid=(nt,),
        in_specs=[
            pl.BlockSpec((TBLK, N_HEAD * HEAD_DIM), lambda i: (i, 0)),
            pl.BlockSpec((N_HEAD * HEAD_DIM, C), lambda i: (0, 0)),
            pl.BlockSpec((TBLK, C), lambda i: (i, 0)),
            pl.BlockSpec((1, C), lambda i: (0, 0)),
            pl.BlockSpec((C, N_EXP), lambda i: (0, 0)),
            pl.BlockSpec((1, N_EXP), lambda i: (0, 0)),
        ],
        out_specs=[pl.BlockSpec((TBLK, C), lambda i: (i, 0)),
                   pl.BlockSpec((TBLK, C), lambda i: (i, 0)),
                   pl.BlockSpec((TBLK, N_EXP), lambda i: (i, 0))],
        out_shape=[jax.ShapeDtypeStruct((T, C), jnp.float32),
                   jax.ShapeDtypeStruct((T, C), jnp.float32),
                   jax.ShapeDtypeStruct((T, N_EXP), jnp.float32)],
    )(y, Wo, xf, ln2_w.reshape(1, C), gateW, expert_bias.reshape(1, N_EXP))


    s, tokb, wb = pl.pallas_call(
        functools.partial(_route_kernel, T=T, NPB=NPB),
        in_specs=[pl.BlockSpec((T, N_EXP), lambda: (0, 0))],
        out_specs=[pl.BlockSpec((2, NPB), lambda: (0, 0)),
                   pl.BlockSpec((NPB, 1, RBLK), lambda: (0, 0, 0)),
                   pl.BlockSpec((NPB, 1, RBLK), lambda: (0, 0, 0))],
        out_shape=[jax.ShapeDtypeStruct((2, NPB), jnp.int32),
                   jax.ShapeDtypeStruct((NPB, 1, RBLK), jnp.float32),
                   jax.ShapeDtypeStruct((NPB, 1, RBLK), jnp.float32)],
    )(logits)

    act = pl.pallas_call(
        functools.partial(_moe_up_kernel, T=T),
        grid_spec=pltpu.PrefetchScalarGridSpec(
            num_scalar_prefetch=1,
            grid=(NPB,),
            in_specs=[
                pl.BlockSpec((1, 1, RBLK), lambda i, s: (i, 0, 0)),
                pl.BlockSpec((T, C), lambda i, s: (0, 0)),
                pl.BlockSpec((1, C, INTER), lambda i, s: (s[0, i], 0, 0)),
                pl.BlockSpec((1, C, INTER), lambda i, s: (s[0, i], 0, 0)),
            ],
            out_specs=pl.BlockSpec((1, RBLK, INTER), lambda i, s: (i, 0, 0)),
        ),
        out_shape=jax.ShapeDtypeStruct((NPB, RBLK, INTER), jnp.float32),
    )(s, tokb, g, gw, uw)

    routed = pl.pallas_call(
        functools.partial(_moe_down_kernel, T=T),
        grid_spec=pltpu.PrefetchScalarGridSpec(
            num_scalar_prefetch=1,
            grid=(NPB,),
            in_specs=[
                pl.BlockSpec((1, 1, RBLK), lambda i, s: (i, 0, 0)),
                pl.BlockSpec((1, 1, RBLK), lambda i, s: (i, 0, 0)),
                pl.BlockSpec((1, RBLK, INTER), lambda i, s: (i, 0, 0)),
                pl.BlockSpec((1, INTER, C), lambda i, s: (s[0, i], 0, 0)),
            ],
            out_specs=pl.BlockSpec((T, C), lambda i, s: (0, 0)),
        ),
        out_shape=jax.ShapeDtypeStruct((T, C), jnp.float32),
    )(s, tokb, wb, act, dw)

    out = pl.pallas_call(
        _final_kernel,
        grid=(nt,),
        in_specs=[
            pl.BlockSpec((TBLK, C), lambda i: (i, 0)),
            pl.BlockSpec((TBLK, C), lambda i: (i, 0)),
            pl.BlockSpec((TBLK, C), lambda i: (i, 0)),
            pl.BlockSpec((C, INTER), lambda i: (0, 0)),
            pl.BlockSpec((C, INTER), lambda i: (0, 0)),
            pl.BlockSpec((INTER, C), lambda i: (0, 0)),
        ],
        out_specs=pl.BlockSpec((TBLK, C), lambda i: (i, 0)),
        out_shape=jax.ShapeDtypeStruct((T, C), jnp.float32),
    )(h, routed, g, sgw, suw, sdw)
    return out.reshape(B, T, C)


# speedup vs baseline: 1.2937x; 1.0202x over previous
import functools

import numpy as np
import jax
import jax.numpy as jnp
from jax.experimental import pallas as pl
from jax.experimental.pallas import tpu as pltpu

N_EMBD = 1024
N_HEAD = 16
HEAD_DIM = 64
KV_LORA = 256
ROPE_DIM = 32
NOPE_DIM = HEAD_DIM - ROPE_DIM
N_EXP = 8
TOP_K = 2
INTER = 2048
THETA = 100000.0

TBLK = 512
RBLK = 256
EPS = 1e-6


def _rms(x, w):
    return x * jax.lax.rsqrt(jnp.mean(x * x, axis=1, keepdims=True) + EPS) * w


def _silu(x):
    return x / (1.0 + jnp.exp(-x))


def _pre_attn_kernel(x_ref, w1_ref, wq_ref, wkva_ref, wkvb_ref, cos_ref,
                     sin_ref, q_ref, k_ref):
    g = _rms(x_ref[...], w1_ref[...])
    q = jnp.dot(g, wq_ref[...], preferred_element_type=jnp.float32)
    ckv = jnp.dot(g, wkva_ref[...], preferred_element_type=jnp.float32)
    latent = ckv[:, :KV_LORA]
    kr = ckv[:, KV_LORA:]
    kn = jnp.dot(latent, wkvb_ref[...], preferred_element_type=jnp.float32)
    C = cos_ref[...]
    S = sin_ref[...]
    di = jax.lax.broadcasted_iota(jnp.int32, (ROPE_DIM, ROPE_DIM), 0)
    dj = jax.lax.broadcasted_iota(jnp.int32, (ROPE_DIM, ROPE_DIM), 1)
    P = ((di ^ 1) == dj).astype(jnp.float32)

    def rope(v):
        return v * C + jnp.dot(v, P, preferred_element_type=jnp.float32) * S

    kr2 = rope(kr)
    qp = []
    kp = []
    for h in range(N_HEAD):
        qh = q[:, h * HEAD_DIM:(h + 1) * HEAD_DIM]
        qp.append(jnp.concatenate(
            [qh[:, :NOPE_DIM], rope(qh[:, NOPE_DIM:])],
            axis=1).reshape(1, TBLK, HEAD_DIM))
        kp.append(jnp.concatenate(
            [kn[:, h * NOPE_DIM:(h + 1) * NOPE_DIM], kr2],
            axis=1).reshape(1, TBLK, HEAD_DIM))
    q_ref[...] = jnp.concatenate(qp, axis=0)
    k_ref[...] = jnp.concatenate(kp, axis=0)


def _attn_kernel(q_ref, k_ref, o_ref, *, T):
    t = pl.program_id(1)
    q = q_ref[0]
    scale = 1.0 / np.float32(np.sqrt(HEAD_DIM))
    rowi = jax.lax.broadcasted_iota(jnp.int32, (TBLK, TBLK), 0) + t * TBLK
    coli = jax.lax.broadcasted_iota(jnp.int32, (TBLK, TBLK), 1)

    def body(j, carry):
        m0, l0, acc0 = carry
        ks = k_ref[0, pl.ds(j * TBLK, TBLK), :]
        s = jax.lax.dot_general(q, ks, (((1,), (1,)), ((), ())),
                                preferred_element_type=jnp.float32) * scale
        s = jnp.where(coli + j * TBLK <= rowi, s, jnp.float32(-1e9))
        m1 = jnp.maximum(m0, jnp.max(s, axis=1, keepdims=True))
        alpha = jnp.exp(m0 - m1)
        p = jnp.exp(s - m1)
        l1 = l0 * alpha + jnp.sum(p, axis=1, keepdims=True)
        acc1 = acc0 * alpha + jnp.dot(p, ks,
                                      preferred_element_type=jnp.float32)
        return m1, l1, acc1

    m0 = jnp.full((TBLK, 1), jnp.float32(-1e30))
    l0 = jnp.zeros((TBLK, 1), jnp.float32)
    acc0 = jnp.zeros((TBLK, HEAD_DIM), jnp.float32)
    m, l, acc = jax.lax.fori_loop(0, t + 1, body, (m0, l0, acc0))
    o_ref[0] = acc * (1.0 / l)


def _post_attn_kernel(y_ref, wo_ref, x_ref, w2_ref, gate_ref, bias_ref,
                      h_ref, g_ref, logit_ref):
    h = x_ref[...] + jnp.dot(y_ref[...], wo_ref[...],
                             preferred_element_type=jnp.float32)
    h_ref[...] = h
    g = _rms(h, w2_ref[...])
    g_ref[...] = g
    logit_ref[...] = (jnp.dot(g, gate_ref[...],
                              preferred_element_type=jnp.float32)
                      + bias_ref[...])


def _route_kernel(logit_ref, s_ref, tok_ref, w_ref, *, T, NPB):
    lg = logit_ref[...]
    m = jnp.max(lg, axis=1, keepdims=True)
    ex = jnp.exp(lg - m)
    probs = ex / jnp.sum(ex, axis=1, keepdims=True)
    io8 = jax.lax.broadcasted_iota(jnp.int32, (T, N_EXP), 1).astype(jnp.float32)
    m1 = jnp.max(probs, axis=1, keepdims=True)
    a1 = jnp.min(jnp.where(probs == m1, io8, jnp.float32(N_EXP)),
                 axis=1, keepdims=True)
    oh1 = (io8 == a1).astype(jnp.float32)
    p2 = jnp.where(oh1 > 0, jnp.float32(-1.0), probs)
    m2 = jnp.max(p2, axis=1, keepdims=True)
    a2 = jnp.min(jnp.where(p2 == m2, io8, jnp.float32(N_EXP)),
                 axis=1, keepdims=True)
    oh2 = (io8 == a2).astype(jnp.float32)
    den = m1 + m2
    w1 = m1 / den
    w2 = m2 / den

    CH = 512
    ci = jax.lax.broadcasted_iota(jnp.int32, (CH, CH), 0)
    cj = jax.lax.broadcasted_iota(jnp.int32, (CH, CH), 1)
    tril = (cj < ci).astype(jnp.float32)

    def exclcum(oh):
        parts = []
        carry = jnp.zeros((1, N_EXP), jnp.float32)
        for c in range(T // CH):
            blk = oh[c * CH:(c + 1) * CH, :]
            parts.append(jnp.dot(tril, blk,
                                 preferred_element_type=jnp.float32) + carry)
            carry = carry + jnp.sum(blk, axis=0, keepdims=True)
        return jnp.concatenate(parts, axis=0), carry

    pos1, tot1 = exclcum(oh1)
    pos2, tot2 = exclcum(oh2)
    counts = tot1 + tot2
    pcnt = jnp.floor((counts + (RBLK - 1)) * (1.0 / RBLK)) * RBLK
    e8i = jax.lax.broadcasted_iota(jnp.int32, (N_EXP, N_EXP), 0)
    e8j = jax.lax.broadcasted_iota(jnp.int32, (N_EXP, N_EXP), 1)
    m8 = (e8i < e8j).astype(jnp.float32)
    pad_off = jnp.dot(pcnt, m8, preferred_element_type=jnp.float32)

    rank1 = jnp.sum(pos1 * oh1, axis=1, keepdims=True)
    rank2 = jnp.sum((pos2 + tot1) * oh2, axis=1, keepdims=True)
    dest1 = jnp.sum(pad_off * oh1, axis=1, keepdims=True) + rank1
    dest2 = jnp.sum(pad_off * oh2, axis=1, keepdims=True) + rank2

    tokc = jax.lax.broadcasted_iota(jnp.int32, (T, 1), 0).astype(jnp.float32)
    riota = jax.lax.broadcasted_iota(jnp.int32, (1, RBLK), 1)

    def body(i, _):
        r = (riota + i * RBLK).astype(jnp.float32)
        mask1 = (dest1 == r).astype(jnp.float32)
        mask2 = (dest2 == r).astype(jnp.float32)
        tok_ref[pl.ds(i, 1)] = jnp.sum(
            mask1 * tokc + mask2 * tokc, axis=0,
            keepdims=True).reshape(1, 1, RBLK)
        w_ref[pl.ds(i, 1)] = jnp.sum(
            mask1 * w1 + mask2 * w2, axis=0,
            keepdims=True).reshape(1, 1, RBLK)
        return 0

    jax.lax.fori_loop(0, NPB, body, 0)

    endb = (pad_off + pcnt) * (1.0 / RBLK)
    eye8 = (e8i == e8j).astype(jnp.float32)
    endb_c = jax.lax.dot_general(eye8, endb, (((1,), (1,)), ((), ())),
                                 preferred_element_type=jnp.float32)
    ib = jax.lax.broadcasted_iota(jnp.int32, (1, NPB), 1).astype(jnp.float32)
    cnt = jnp.sum((endb_c <= ib).astype(jnp.float32), axis=0, keepdims=True)
    eblk = jnp.minimum(cnt, jnp.float32(N_EXP - 1))
    nact = jnp.sum(pcnt, axis=1, keepdims=True) * (1.0 / RBLK)
    nact = jnp.broadcast_to(nact, (1, NPB))
    s_ref[...] = jnp.concatenate([eblk, nact], axis=0).astype(jnp.int32)


def _moe_up_kernel(s_ref, tok_ref, g_ref, gw_ref, uw_ref, act_ref, *, T):
    i = pl.program_id(0)

    @pl.when(i < s_ref[1, 0])
    def _():
        tok = tok_ref[...].reshape(1, RBLK)
        ti = jax.lax.broadcasted_iota(jnp.int32,
                                      (T, RBLK), 0).astype(jnp.float32)
        G = (tok == ti).astype(jnp.float32)
        xg = jax.lax.dot_general(G, g_ref[...], (((0,), (0,)), ((), ())),
                                 preferred_element_type=jnp.float32)
        h1 = jnp.dot(xg, gw_ref[0], preferred_element_type=jnp.float32)
        h2 = jnp.dot(xg, uw_ref[0], preferred_element_type=jnp.float32)
        act_ref[...] = (_silu(h1) * h2).astype(
            jnp.bfloat16).reshape(1, RBLK, INTER)


def _moe_down_kernel(s_ref, tok_ref, w_ref, act_ref, dw_ref, out_ref, *, T):
    i = pl.program_id(0)

    @pl.when(i == 0)
    def _():
        out_ref[...] = jnp.zeros_like(out_ref)

    @pl.when(i < s_ref[1, 0])
    def _():
        act = act_ref[...].reshape(RBLK, INTER).astype(jnp.float32)
        y = jnp.dot(act, dw_ref[0], preferred_element_type=jnp.float32)
        tok = tok_ref[...].reshape(1, RBLK)
        wrow = w_ref[...].reshape(1, RBLK)
        ti = jax.lax.broadcasted_iota(jnp.int32,
                                      (T, RBLK), 0).astype(jnp.float32)
        Gw = (tok == ti).astype(jnp.float32) * wrow
        out_ref[...] += jnp.dot(Gw, y, preferred_element_type=jnp.float32)


def _final_kernel(h_ref, r_ref, g_ref, sgw_ref, suw_ref, sdw_ref, o_ref):
    g = g_ref[...]
    h1 = jnp.dot(g, sgw_ref[...], preferred_element_type=jnp.float32)
    h2 = jnp.dot(g, suw_ref[...], preferred_element_type=jnp.float32)
    act = _silu(h1) * h2
    o_ref[...] = (h_ref[...] + r_ref[...]
                  + jnp.dot(act, sdw_ref[...],
                            preferred_element_type=jnp.float32))


def kernel(x, ln1_w, ln2_w, Wq, Wkva, Wkvb, Wo, gateW, expert_bias,
           gw, uw, dw, sgw, suw, sdw):
    B, T, C = x.shape
    xf = x.reshape(T, C)
    nt = T // TBLK
    NPB = (TOP_K * T) // RBLK + N_EXP

    freqs = 1.0 / (THETA ** (np.arange(0, ROPE_DIM, 2, dtype=np.float32)
                             / ROPE_DIM))
    f = np.outer(np.arange(T, dtype=np.float32), freqs)
    cosv = np.cos(f).astype(np.float32)
    sinv = np.sin(f).astype(np.float32)
    Ct = np.empty((T, ROPE_DIM), np.float32)
    St = np.empty((T, ROPE_DIM), np.float32)
    Ct[:, 0::2] = cosv
    Ct[:, 1::2] = cosv
    St[:, 0::2] = -sinv
    St[:, 1::2] = sinv
    Ct = jnp.asarray(Ct)
    St = jnp.asarray(St)

    qf, kf = pl.pallas_call(
        _pre_attn_kernel,
        grid=(nt,),
        in_specs=[
            pl.BlockSpec((TBLK, C), lambda i: (i, 0)),
            pl.BlockSpec((1, C), lambda i: (0, 0)),
            pl.BlockSpec((C, N_HEAD * HEAD_DIM), lambda i: (0, 0)),
            pl.BlockSpec((C, KV_LORA + ROPE_DIM), lambda i: (0, 0)),
            pl.BlockSpec((KV_LORA, N_HEAD * NOPE_DIM), lambda i: (0, 0)),
            pl.BlockSpec((TBLK, ROPE_DIM), lambda i: (i, 0)),
            pl.BlockSpec((TBLK, ROPE_DIM), lambda i: (i, 0)),
        ],
        out_specs=[pl.BlockSpec((N_HEAD, TBLK, HEAD_DIM), lambda i: (0, i, 0)),
                   pl.BlockSpec((N_HEAD, TBLK, HEAD_DIM), lambda i: (0, i, 0))],
        out_shape=[jax.ShapeDtypeStruct((N_HEAD, T, HEAD_DIM),
                                        jnp.float32)] * 2,
    )(xf, ln1_w.reshape(1, C), Wq, Wkva, Wkvb, Ct, St)

    yT = pl.pallas_call(
        functools.partial(_attn_kernel, T=T),
        grid=(N_HEAD, nt),
        in_specs=[
            pl.BlockSpec((1, TBLK, HEAD_DIM), lambda h, t: (h, t, 0)),
            pl.BlockSpec((1, T, HEAD_DIM), lambda h, t: (h, 0, 0)),
        ],
        out_specs=pl.BlockSpec((1, TBLK, HEAD_DIM), lambda h, t: (h, t, 0)),
        out_shape=jax.ShapeDtypeStruct((N_HEAD, T, HEAD_DIM), jnp.float32),
    )(qf, kf)
    y = yT.transpose(1, 0, 2).reshape(T, N_HEAD * HEAD_DIM)

    h, g, logits = pl.pallas_call(
        _post_attn_kernel,
        grid=(nt,),
        in_specs=[
            pl.BlockSpec((TBLK, N_HEAD * HEAD_DIM), lambda i: (i, 0)),
            pl.BlockSpec((N_HEAD * HEAD_DIM, C), lambda i: (0, 0)),
            pl.BlockSpec((TBLK, C), lambda i: (i, 0)),
            pl.BlockSpec((1, C), lambda i: (0, 0)),
            pl.BlockSpec((C, N_EXP), lambda i: (0, 0)),
            pl.BlockSpec((1, N_EXP), lambda i: (0, 0)),
        ],
        out_specs=[pl.BlockSpec((TBLK, C), lambda i: (i, 0)),
                   pl.BlockSpec((TBLK, C), lambda i: (i, 0)),
                   pl.BlockSpec((TBLK, N_EXP), lambda i: (i, 0))],
        out_shape=[jax.ShapeDtypeStruct((T, C), jnp.float32),
                   jax.ShapeDtypeStruct((T, C), jnp.float32),
                   jax.ShapeDtypeStruct((T, N_EXP), jnp.float32)],
    )(y, Wo, xf, ln2_w.reshape(1, C), gateW, expert_bias.reshape(1, N_EXP))


    s, tokb, wb = pl.pallas_call(
        functools.partial(_route_kernel, T=T, NPB=NPB),
        in_specs=[pl.BlockSpec((T, N_EXP), lambda: (0, 0))],
        out_specs=[pl.BlockSpec((2, NPB), lambda: (0, 0)),
                   pl.BlockSpec((NPB, 1, RBLK), lambda: (0, 0, 0)),
                   pl.BlockSpec((NPB, 1, RBLK), lambda: (0, 0, 0))],
        out_shape=[jax.ShapeDtypeStruct((2, NPB), jnp.int32),
                   jax.ShapeDtypeStruct((NPB, 1, RBLK), jnp.float32),
                   jax.ShapeDtypeStruct((NPB, 1, RBLK), jnp.float32)],
    )(logits)

    act = pl.pallas_call(
        functools.partial(_moe_up_kernel, T=T),
        grid_spec=pltpu.PrefetchScalarGridSpec(
            num_scalar_prefetch=1,
            grid=(NPB,),
            in_specs=[
                pl.BlockSpec((1, 1, RBLK), lambda i, s: (i, 0, 0)),
                pl.BlockSpec((T, C), lambda i, s: (0, 0)),
                pl.BlockSpec((1, C, INTER), lambda i, s: (s[0, i], 0, 0)),
                pl.BlockSpec((1, C, INTER), lambda i, s: (s[0, i], 0, 0)),
            ],
            out_specs=pl.BlockSpec((1, RBLK, INTER), lambda i, s: (i, 0, 0)),
        ),
        out_shape=jax.ShapeDtypeStruct((NPB, RBLK, INTER), jnp.bfloat16),
    )(s, tokb, g, gw, uw)

    routed = pl.pallas_call(
        functools.partial(_moe_down_kernel, T=T),
        grid_spec=pltpu.PrefetchScalarGridSpec(
            num_scalar_prefetch=1,
            grid=(NPB,),
            in_specs=[
                pl.BlockSpec((1, 1, RBLK), lambda i, s: (i, 0, 0)),
                pl.BlockSpec((1, 1, RBLK), lambda i, s: (i, 0, 0)),
                pl.BlockSpec((1, RBLK, INTER), lambda i, s: (i, 0, 0)),
                pl.BlockSpec((1, INTER, C), lambda i, s: (s[0, i], 0, 0)),
            ],
            out_specs=pl.BlockSpec((T, C), lambda i, s: (0, 0)),
        ),
        out_shape=jax.ShapeDtypeStruct((T, C), jnp.float32),
    )(s, tokb, wb, act, dw)

    out = pl.pallas_call(
        _final_kernel,
        grid=(nt,),
        in_specs=[
            pl.BlockSpec((TBLK, C), lambda i: (i, 0)),
            pl.BlockSpec((TBLK, C), lambda i: (i, 0)),
            pl.BlockSpec((TBLK, C), lambda i: (i, 0)),
            pl.BlockSpec((C, INTER), lambda i: (0, 0)),
            pl.BlockSpec((C, INTER), lambda i: (0, 0)),
            pl.BlockSpec((INTER, C), lambda i: (0, 0)),
        ],
        out_specs=pl.BlockSpec((TBLK, C), lambda i: (i, 0)),
        out_shape=jax.ShapeDtypeStruct((T, C), jnp.float32),
    )(h, routed, g, sgw, suw, sdw)
    return out.reshape(B, T, C)
